# Initial kernel scaffold; baseline (speedup 1.0000x reference)
import jax
import jax.numpy as jnp
from jax.experimental import pallas as pl


def kernel(boxes, scores, hidden_states, labels):
    raise NotImplementedError("write your pallas kernel here")


# speedup vs baseline: 11.1612x; 11.1612x over previous
import functools

import jax
import jax.numpy as jnp
from jax import lax
from jax.experimental import pallas as pl
from jax.experimental.pallas import tpu as pltpu

_N = 5000
_ROWS = 40
_LANE = 128
_NPAD = _ROWS * _LANE
_DH = 256
_NSEL = 15
_THR = 0.2
_IOU = 0.5
_HUMAN = 0


def _body(sx0, sy0, sx1, sy1, sar, ssc, shu, spe,
          ox0, oy0, ox1, oy1, osc,
          out, keep, selh, selo):
    lane = lax.broadcasted_iota(jnp.int32, (1, _LANE), 1)

    def ext(ref, r, c):
        row = ref[pl.ds(r, 1), :]
        z = jnp.zeros_like(row)
        return jnp.sum(jnp.where(lane == c, row, z))

    def put(ref, r, c, val):
        row = ref[pl.ds(r, 1), :]
        ref[pl.ds(r, 1), :] = jnp.where(lane == c, val, row)

    keep[...] = jnp.zeros((_ROWS, _LANE), jnp.float32)
    selh[...] = jnp.zeros((8, _LANE), jnp.int32)
    selo[...] = jnp.zeros((8, _LANE), jnp.int32)

    def cond(c):
        return c[3] == 1

    def body(c):
        t, nh, no, _ = c
        r = lax.shift_right_logical(t, 7)
        cl = jnp.bitwise_and(t, _LANE - 1)
        act = ext(ssc, r, cl) >= _THR
        tx0 = ext(sx0, r, cl)
        ty0 = ext(sy0, r, cl)
        tx1 = ext(sx1, r, cl)
        ty1 = ext(sy1, r, cl)
        ta = ext(sar, r, cl)

        def rowstep(k, s):
            jx0 = sx0[pl.ds(k, 1), :]
            jy0 = sy0[pl.ds(k, 1), :]
            jx1 = sx1[pl.ds(k, 1), :]
            jy1 = sy1[pl.ds(k, 1), :]
            ja = sar[pl.ds(k, 1), :]
            kb = keep[pl.ds(k, 1), :]
            w = jnp.maximum(jnp.minimum(jx1, tx1) - jnp.maximum(jx0, tx0), 0.0)
            h = jnp.maximum(jnp.minimum(jy1, ty1) - jnp.maximum(jy0, ty0), 0.0)
            inter = w * h
            iou = inter / (ta + ja - inter + 1e-9)
            jidx = lane + k * _LANE
            hit = (iou > _IOU) & (kb > 0.5) & (jidx < t)
            return s + jnp.sum(jnp.where(hit, 1.0, 0.0))

        supp = lax.fori_loop(0, r + 1, rowstep, jnp.float32(0.0))
        kept = act & (supp < 0.5)

        @pl.when(kept)
        def _():
            put(keep, r, cl, 1.0)

        ish = ext(shu, r, cl) > 0.5
        pt = ext(spe, r, cl)
        takeh = kept & ish & (nh < _NSEL)
        takeo = kept & (~ish) & (no < _NSEL)

        @pl.when(takeh)
        def _():
            put(selh, 0, nh, pt)

        @pl.when(takeo)
        def _():
            put(selo, 0, no, pt)

        nh2 = nh + jnp.where(takeh, 1, 0)
        no2 = no + jnp.where(takeo, 1, 0)
        t2 = t + 1
        go = jnp.where(
            act & (t2 < _NPAD) & ((nh2 < _NSEL) | (no2 < _NSEL)), 1, 0)
        return (t2, nh2, no2, go)

    _, nh, no, _ = lax.while_loop(
        cond, body,
        (jnp.int32(0), jnp.int32(0), jnp.int32(0), jnp.int32(1)))

    def mk_fill(sel, cnt_valid):
        def fill(o, cnt):
            row = sel[pl.ds(0, 1), :].astype(jnp.float32)
            present = jnp.sum(jnp.where(
                (row == o) & (lane < cnt_valid), 1.0, 0.0)) > 0.5
            take = (~present) & (cnt < _NSEL - cnt_valid)

            @pl.when(take)
            def _():
                put(sel, 0, cnt_valid + cnt, o)

            return cnt + jnp.where(take, 1, 0)
        return fill

    lax.fori_loop(0, 2 * _NSEL, mk_fill(selh, nh), jnp.int32(0))
    lax.fori_loop(0, 2 * _NSEL, mk_fill(selo, no), jnp.int32(0))

    for s in range(2 * _NSEL):
        if s < _NSEL:
            oi = ext(selh, 0, s)
        else:
            oi = ext(selo, 0, s - _NSEL)
        r = lax.shift_right_logical(oi, 7)
        cl = jnp.bitwise_and(oi, _LANE - 1)
        sc = ext(osc, r, cl)
        sig = 1.0 / (1.0 + jnp.exp(-sc))
        vals = (jnp.where(lane == 0, oi.astype(jnp.float32), 0.0)
                + jnp.where(lane == 1, sig, 0.0)
                + jnp.where(lane == 2, ext(ox0, r, cl), 0.0)
                + jnp.where(lane == 3, ext(oy0, r, cl), 0.0)
                + jnp.where(lane == 4, ext(ox1, r, cl), 0.0)
                + jnp.where(lane == 5, ext(oy1, r, cl), 0.0))
        out[pl.ds(s, 1), :] = vals


def _nms_call(*args):
    f32 = jnp.float32
    return pl.pallas_call(
        _body,
        out_shape=jax.ShapeDtypeStruct((2 * _NSEL, _LANE), f32),
        scratch_shapes=[
            pltpu.VMEM((_ROWS, _LANE), f32),
            pltpu.VMEM((8, _LANE), jnp.int32),
            pltpu.VMEM((8, _LANE), jnp.int32),
        ],
    )(*args)


def kernel(boxes, scores, hidden_states, labels):
    f32, i32 = jnp.float32, jnp.int32
    mc = jnp.max(boxes)
    ob = boxes + (labels.astype(f32) * (mc + 1.0))[:, None]
    order = jnp.argsort(-scores)
    sb = ob[order]

    def pad2(x, v):
        return jnp.concatenate(
            [x, jnp.full((_NPAD - _N,), v, x.dtype)]).reshape(_ROWS, _LANE)

    sx0 = pad2(sb[:, 0], 0.0)
    sy0 = pad2(sb[:, 1], 0.0)
    sx1 = pad2(sb[:, 2], 0.0)
    sy1 = pad2(sb[:, 3], 0.0)
    sar = pad2((sb[:, 2] - sb[:, 0]) * (sb[:, 3] - sb[:, 1]), 0.0)
    ssc = pad2(scores[order], -1.0)
    shu = pad2((labels[order] == _HUMAN).astype(i32), 0)
    spe = pad2(order.astype(i32), 0)
    ox0 = pad2(boxes[:, 0], 0.0)
    oy0 = pad2(boxes[:, 1], 0.0)
    ox1 = pad2(boxes[:, 2], 0.0)
    oy1 = pad2(boxes[:, 3], 0.0)
    osc = pad2(scores, 0.0)

    res = _nms_call(sx0, sy0, sx1, sy1, sar, ssc, shu, spe,
                    ox0, oy0, ox1, oy1, osc)
    idx = res[:, 0].astype(i32)
    sig = res[:, 1]
    bx = res[:, 2:6]
    return jnp.concatenate([hidden_states[idx] * sig[:, None], bx], axis=1)
